# baseline (speedup 1.0000x reference)
import jax
import jax.numpy as jnp
from jax import lax
from jax.experimental import pallas as pl
from jax.experimental.pallas import tpu as pltpu
from jax.experimental.pallas import tpu_sc as plsc

VOCAB = 100000
B = 4096
HIST = 50
D = 64
U = 128
ROW_W = 128
TCB = 1024
HALF = 50176

NC = 2
NS = 16
NW = NC * NS
ROWS_PER_W = B // NW
NBUF = 4

IDB = 512
GLEN = 56
W0_COL = GLEN
W1_COL = GLEN + 1
W0_BITS = 31


def _sq_body(lo_ref, hi_ref, o_ref):
  r = lax.broadcasted_iota(jnp.int32, (D, D), 0)
  c = lax.broadcasted_iota(jnp.int32, (D, D), 1)
  eye = (r == c).astype(jnp.bfloat16)
  dn = (((0,), (0,)), ((), ()))
  lo = lo_ref[...]
  hi = hi_ref[...]
  o_ref[:, 0:D] = lax.dot_general((lo * lo).astype(jnp.bfloat16), eye, dn,
                                  preferred_element_type=jnp.float32)
  o_ref[:, D:ROW_W] = lax.dot_general((hi * hi).astype(jnp.bfloat16), eye, dn,
                                      preferred_element_type=jnp.float32)


def _sq_table(emb_t):
  return pl.pallas_call(
      _sq_body,
      out_shape=jax.ShapeDtypeStruct((HALF, ROW_W), jnp.float32),
      grid=(HALF // TCB,),
      in_specs=[
          pl.BlockSpec((D, TCB), lambda j: (0, j)),
          pl.BlockSpec((D, TCB), lambda j: (0, j + HALF // TCB)),
      ],
      out_specs=pl.BlockSpec((TCB, ROW_W), lambda j: (j, 0)),
  )(emb_t, emb_t)


def _ids_body(x_ref, o_ref):
  x = x_ref[...]
  hi = (x >= HALF).astype(jnp.int32)
  l = lax.broadcasted_iota(jnp.int32, (1, HIST), 1)
  sh0 = jnp.where(l < W0_BITS, jnp.minimum(l, W0_BITS - 1), 0)
  m0 = jnp.where(l < W0_BITS, 1, 0)
  sh1 = jnp.where(l >= W0_BITS, l - W0_BITS, 0)
  m1 = jnp.where(l >= W0_BITS, 1, 0)
  w0 = jnp.sum(hi * m0 * jnp.left_shift(1, sh0), axis=1, keepdims=True)
  w1 = jnp.sum(hi * m1 * jnp.left_shift(1, sh1), axis=1, keepdims=True)
  o_ref[:, 0:HIST] = jnp.where(hi == 1, x - HALF, x)
  o_ref[:, HIST:GLEN] = jnp.zeros((IDB, GLEN - HIST), jnp.int32)
  o_ref[:, W0_COL:W0_COL + 1] = w0
  o_ref[:, W1_COL:W1_COL + 1] = w1
  o_ref[:, W1_COL + 1:ROW_W] = jnp.zeros((IDB, ROW_W - W1_COL - 1), jnp.int32)


def _prep_ids(ids):
  return pl.pallas_call(
      _ids_body,
      out_shape=jax.ShapeDtypeStruct((B, ROW_W), jnp.int32),
      grid=(B // IDB,),
      in_specs=[pl.BlockSpec((IDB, HIST), lambda i: (i, 0))],
      out_specs=pl.BlockSpec((IDB, ROW_W), lambda i: (i, 0)),
  )(ids)


def _sc_pooled_body(table_hbm, pidx_hbm, out_hbm, pidx_v, rows_v, pooled_v,
                    *sems):
  cid = lax.axis_index("c")
  sid = lax.axis_index("s")
  wid = sid * NC + cid
  base = wid * ROWS_PER_W

  pltpu.sync_copy(pidx_hbm.at[pl.ds(base, ROWS_PER_W)], pidx_v)

  for b in range(NBUF):
    pltpu.make_async_copy(
        table_hbm.at[pidx_v.at[b, pl.ds(0, GLEN)]], rows_v.at[b],
        sems[b]).start()

  def group_body(g, carry):
    for b in range(NBUF):
      j = g * NBUF + b
      pltpu.make_async_copy(
          table_hbm.at[pidx_v.at[j, pl.ds(0, GLEN)]], rows_v.at[b],
          sems[b]).wait()
      hv = pidx_v[j, pl.ds(48, 16)]
      w0 = hv[W0_COL - 48]
      w1 = hv[W1_COL - 48]
      acc = [jnp.zeros((16,), jnp.float32) for _ in range(4)]
      for l in range(HIST):
        if l < W0_BITS:
          bit = lax.shift_right_logical(w0, l) & 1
        else:
          bit = lax.shift_right_logical(w1, l - W0_BITS) & 1
        off = bit * D
        for v in range(4):
          acc[v] = acc[v] + rows_v[b, l, pl.ds(off + v * 16, 16)]
      for v in range(4):
        pooled_v[j, pl.ds(v * 16, 16)] = acc[v]

      @pl.when(j + NBUF < ROWS_PER_W)
      def _refill():
        pltpu.make_async_copy(
            table_hbm.at[pidx_v.at[j + NBUF, pl.ds(0, GLEN)]], rows_v.at[b],
            sems[b]).start()
    return carry

  lax.fori_loop(0, ROWS_PER_W // NBUF, group_body, 0)

  pltpu.sync_copy(pooled_v, out_hbm.at[pl.ds(base, ROWS_PER_W)])


def _sc_pooled(sq_table, pidx):
  mesh = plsc.VectorSubcoreMesh(core_axis_name="c", subcore_axis_name="s")
  return pl.kernel(
      _sc_pooled_body,
      out_type=jax.ShapeDtypeStruct((B, ROW_W), jnp.float32),
      mesh=mesh,
      compiler_params=pltpu.CompilerParams(use_tc_tiling_on_sc=False),
      scratch_types=[
          pltpu.VMEM((ROWS_PER_W, ROW_W), jnp.int32),
          pltpu.VMEM((NBUF, GLEN, ROW_W), jnp.float32),
          pltpu.VMEM((ROWS_PER_W, ROW_W), jnp.float32),
      ] + [pltpu.SemaphoreType.DMA] * NBUF,
  )(sq_table, pidx)


def _mm_body(p_ref, k_ref, o_ref):
  o_ref[...] = jnp.dot(p_ref[:, 0:D], k_ref[...],
                       preferred_element_type=jnp.float32)


def _dense(pooled, w):
  return pl.pallas_call(
      _mm_body,
      out_shape=jax.ShapeDtypeStruct((B, U), jnp.float32),
  )(pooled, w)


@jax.jit
def _run(input_ids, embedding, w):
  sq = _sq_table(embedding.T)
  pidx = _prep_ids(input_ids)
  pooled = _sc_pooled(sq, pidx)
  return _dense(pooled, w)


def kernel(input_ids, embedding, kernel):
  return _run(input_ids, embedding, kernel)


# speedup vs baseline: 8.1740x; 8.1740x over previous
import jax
import jax.numpy as jnp
from jax import lax
from jax.experimental import pallas as pl
from jax.experimental.pallas import tpu as pltpu
from jax.experimental.pallas import tpu_sc as plsc

B = 4096
HIST = 50
D = 64
U = 128
ROW_W = 128

NC = 2
NS = 16
NW = NC * NS
ROWS_PER_W = B // NW
NBUF = 8


def _sc_pooled_body(table_hbm, ids_hbm, out_hbm, idx_v, rows_v, pooled_v,
                    *sems):
  cid = lax.axis_index("c")
  sid = lax.axis_index("s")
  wid = sid * NC + cid
  base = wid * ROWS_PER_W

  pltpu.sync_copy(ids_hbm.at[pl.ds(base, ROWS_PER_W)], idx_v)

  for b in range(NBUF):
    pltpu.make_async_copy(
        table_hbm.at[idx_v.at[b]], rows_v.at[b], sems[b]).start()

  def group_body(g, carry):
    for b in range(NBUF):
      j = g * NBUF + b
      pltpu.make_async_copy(
          table_hbm.at[idx_v.at[j]], rows_v.at[b], sems[b]).wait()
      acc = [jnp.zeros((16,), jnp.float32) for _ in range(4)]
      for l in range(HIST):
        for v in range(4):
          x = rows_v[b, l, pl.ds(v * 16, 16)]
          acc[v] = acc[v] + x * x
      for v in range(4):
        pooled_v[j, pl.ds(v * 16, 16)] = acc[v]

      @pl.when(j + NBUF < ROWS_PER_W)
      def _refill():
        pltpu.make_async_copy(
            table_hbm.at[idx_v.at[j + NBUF]], rows_v.at[b], sems[b]).start()
    return carry

  lax.fori_loop(0, ROWS_PER_W // NBUF, group_body, 0)

  pltpu.sync_copy(pooled_v, out_hbm.at[pl.ds(base, ROWS_PER_W)])


def _sc_pooled(embedding, ids):
  mesh = plsc.VectorSubcoreMesh(core_axis_name="c", subcore_axis_name="s")
  return pl.kernel(
      _sc_pooled_body,
      out_type=jax.ShapeDtypeStruct((B, ROW_W), jnp.float32),
      mesh=mesh,
      compiler_params=pltpu.CompilerParams(use_tc_tiling_on_sc=False),
      scratch_types=[
          pltpu.VMEM((ROWS_PER_W, HIST), jnp.int32),
          pltpu.VMEM((NBUF, HIST, D), jnp.float32),
          pltpu.VMEM((ROWS_PER_W, ROW_W), jnp.float32),
      ] + [pltpu.SemaphoreType.DMA] * NBUF,
  )(embedding, ids)


def _mm_body(p_ref, k_ref, o_ref):
  o_ref[...] = jnp.dot(p_ref[:, 0:D], k_ref[...],
                       preferred_element_type=jnp.float32)


def _dense(pooled, w):
  return pl.pallas_call(
      _mm_body,
      out_shape=jax.ShapeDtypeStruct((B, U), jnp.float32),
  )(pooled, w)


@jax.jit
def _run(input_ids, embedding, w):
  pooled = _sc_pooled(embedding, input_ids)
  return _dense(pooled, w)


def kernel(input_ids, embedding, kernel):
  return _run(input_ids, embedding, kernel)


# speedup vs baseline: 8.9350x; 1.0931x over previous
import jax
import jax.numpy as jnp
from jax import lax
from jax.experimental import pallas as pl
from jax.experimental.pallas import tpu as pltpu
from jax.experimental.pallas import tpu_sc as plsc

B = 4096
HIST = 50
D = 64
U = 128
NC = 2
NS = 16
NW = NC * NS
ROWS_PER_W = B // NW
NBUF = 4


def _sc_pooled_body(table_hbm, ids_hbm, out_hbm, idx_v, rows_v, pooled_v,
                    *sems):
  cid = lax.axis_index("c")
  sid = lax.axis_index("s")
  wid = sid * NC + cid
  base = wid * ROWS_PER_W

  pltpu.sync_copy(ids_hbm.at[pl.ds(base, ROWS_PER_W)], idx_v)

  for b in range(NBUF):
    pltpu.make_async_copy(
        table_hbm.at[idx_v.at[b]], rows_v.at[b], sems[b]).start()

  def group_body(g, carry):
    for b in range(NBUF):
      j = g * NBUF + b
      pltpu.make_async_copy(
          table_hbm.at[idx_v.at[j]], rows_v.at[b], sems[b]).wait()
      acc = [jnp.zeros((16,), jnp.float32) for _ in range(4)]
      for l in range(HIST):
        for v in range(4):
          x = rows_v[b, l, pl.ds(v * 16, 16)]
          acc[v] = acc[v] + x * x
      for v in range(4):
        pooled_v[j, pl.ds(v * 16, 16)] = acc[v]

      @pl.when(j + NBUF < ROWS_PER_W)
      def _refill():
        pltpu.make_async_copy(
            table_hbm.at[idx_v.at[j + NBUF]], rows_v.at[b], sems[b]).start()
    return carry

  lax.fori_loop(0, ROWS_PER_W // NBUF, group_body, 0)

  pltpu.sync_copy(pooled_v, out_hbm.at[pl.ds(base, ROWS_PER_W)])


def _sc_pooled(embedding, ids):
  mesh = plsc.VectorSubcoreMesh(core_axis_name="c", subcore_axis_name="s")
  return pl.kernel(
      _sc_pooled_body,
      out_type=jax.ShapeDtypeStruct((B, D), jnp.float32),
      mesh=mesh,
      compiler_params=pltpu.CompilerParams(use_tc_tiling_on_sc=False),
      scratch_types=[
          pltpu.VMEM((ROWS_PER_W, HIST), jnp.int32),
          pltpu.VMEM((NBUF, HIST, D), jnp.float32),
          pltpu.VMEM((ROWS_PER_W, D), jnp.float32),
      ] + [pltpu.SemaphoreType.DMA] * NBUF,
  )(embedding, ids)


def _mm_body(p_ref, k_ref, o_ref):
  o_ref[...] = jnp.dot(p_ref[...], k_ref[...],
                       preferred_element_type=jnp.float32)


def _dense(pooled, w):
  return pl.pallas_call(
      _mm_body,
      out_shape=jax.ShapeDtypeStruct((B, U), jnp.float32),
  )(pooled, w)


@jax.jit
def _run(input_ids, embedding, w):
  pooled = _sc_pooled(embedding, input_ids)
  return _dense(pooled, w)


def kernel(input_ids, embedding, kernel):
  return _run(input_ids, embedding, kernel)
